# Initial kernel scaffold; baseline (speedup 1.0000x reference)
import functools

import jax
import jax.numpy as jnp
from jax.experimental import pallas as pl
from jax.experimental.pallas import tpu as pltpu
from jax.experimental.pallas import tpu_sc as plsc

_B = 4096
_L = 50
_DIM = 300
_N = _B * _L
_W = 128


def kernel(inputs, weight):
    idx = inputs.reshape(1, _N)
    mesh = plsc.VectorSubcoreMesh(core_axis_name="c", subcore_axis_name="s")

    @functools.partial(
        pl.kernel,
        out_type=jax.ShapeDtypeStruct((_N, _DIM), weight.dtype),
        mesh=mesh,
    )
    def gather_kernel(w_hbm, i_hbm, o_hbm):
        def body(i_vmem, o_vmem):
            pltpu.sync_copy(w_hbm.at[i_vmem.at[0]], o_vmem)

        pltpu.emit_pipeline(
            body,
            grid=(_N // _W,),
            in_specs=[pl.BlockSpec((1, _W), lambda i: (0, i))],
            out_specs=[pl.BlockSpec((_W, _DIM), lambda i: (i, 0))],
            core_axis_name=("c", "s"),
            dimension_semantics=(pltpu.PARALLEL,),
        )(i_hbm, o_hbm)

    out = gather_kernel(weight, idx)
    return out.reshape(_B, _L, _DIM)


# speedup vs baseline: 1.4943x; 1.4943x over previous
import functools

import jax
import jax.numpy as jnp
from jax import lax
from jax.experimental import pallas as pl
from jax.experimental.pallas import tpu as pltpu
from jax.experimental.pallas import tpu_sc as plsc

_B = 4096
_L = 50
_DIM = 300
_N = _B * _L
_W = 128
_NW = 32
_STEPS = _N // _W // _NW


def kernel(inputs, weight):
    idx = inputs.reshape(_N)
    w_tail = jnp.pad(weight[:, 256:_DIM], ((0, 0), (0, 84)))
    mesh = plsc.VectorSubcoreMesh(core_axis_name="c", subcore_axis_name="s")

    @functools.partial(
        pl.kernel,
        out_type=jax.ShapeDtypeStruct((_N, 384), weight.dtype),
        mesh=mesh,
        scratch_types=[
            pltpu.VMEM((_W,), jnp.int32),
            pltpu.VMEM((_W, 128), jnp.float32),
            pltpu.VMEM((_W, 128), jnp.float32),
            pltpu.VMEM((_W, 128), jnp.float32),
            pltpu.SemaphoreType.DMA,
            pltpu.SemaphoreType.DMA,
            pltpu.SemaphoreType.DMA,
        ],
    )
    def gather_kernel(w_hbm, wt_hbm, i_hbm, o_hbm, idx_v, b1, b2, b3,
                      s1, s2, s3):
        wid = lax.axis_index("s") * 2 + lax.axis_index("c")

        @pl.loop(0, _STEPS)
        def _(c):
            base = (wid * _STEPS + c) * _W
            pltpu.sync_copy(i_hbm.at[pl.ds(base, _W)], idx_v)
            c1 = pltpu.async_copy(w_hbm.at[:, pl.ds(0, 128)].at[idx_v], b1, s1)
            c2 = pltpu.async_copy(w_hbm.at[:, pl.ds(128, 128)].at[idx_v], b2, s2)
            c3 = pltpu.async_copy(wt_hbm.at[idx_v], b3, s3)
            c1.wait()
            c2.wait()
            c3.wait()
            o1 = pltpu.async_copy(b1, o_hbm.at[pl.ds(base, _W), pl.ds(0, 128)], s1)
            o2 = pltpu.async_copy(b2, o_hbm.at[pl.ds(base, _W), pl.ds(128, 128)], s2)
            o3 = pltpu.async_copy(b3, o_hbm.at[pl.ds(base, _W), pl.ds(256, 128)], s3)
            o1.wait()
            o2.wait()
            o3.wait()

    out = gather_kernel(weight, w_tail, idx)
    return out[:, :_DIM].reshape(_B, _L, _DIM)


# speedup vs baseline: 1.6504x; 1.1045x over previous
import functools

import jax
import jax.numpy as jnp
from jax import lax
from jax.experimental import pallas as pl
from jax.experimental.pallas import tpu as pltpu
from jax.experimental.pallas import tpu_sc as plsc

_B = 4096
_L = 50
_DIM = 300
_NW = 32
_BPW = _B // _NW


def kernel(inputs, weight):
    w_tail = jnp.pad(weight[:, 256:_DIM], ((0, 0), (0, 84)))
    mesh = plsc.VectorSubcoreMesh(core_axis_name="c", subcore_axis_name="s")

    @functools.partial(
        pl.kernel,
        out_type=jax.ShapeDtypeStruct((_B, _L, _DIM), weight.dtype),
        mesh=mesh,
        scratch_types=[
            pltpu.VMEM((_L,), jnp.int32),
            pltpu.VMEM((_L, 128), jnp.float32),
            pltpu.VMEM((_L, 128), jnp.float32),
            pltpu.VMEM((_L, 128), jnp.float32),
            pltpu.VMEM((_L, 44), jnp.float32),
            pltpu.SemaphoreType.DMA,
            pltpu.SemaphoreType.DMA,
            pltpu.SemaphoreType.DMA,
        ],
    )
    def gather_kernel(w_hbm, wt_hbm, i_hbm, o_hbm, idx_v, b1, b2, b3, b4,
                      s1, s2, s3):
        wid = lax.axis_index("s") * 2 + lax.axis_index("c")

        @pl.loop(0, _BPW)
        def _(c):
            b = wid * _BPW + c
            pltpu.sync_copy(i_hbm.at[b], idx_v)
            c1 = pltpu.async_copy(w_hbm.at[:, pl.ds(0, 128)].at[idx_v], b1, s1)
            c2 = pltpu.async_copy(w_hbm.at[:, pl.ds(128, 128)].at[idx_v], b2, s2)
            c3 = pltpu.async_copy(wt_hbm.at[idx_v], b3, s3)
            c1.wait()
            c2.wait()
            c3.wait()

            @pl.loop(0, _L)
            def _(l):
                b4[l, pl.ds(0, 16)] = b3[l, pl.ds(0, 16)]
                b4[l, pl.ds(16, 16)] = b3[l, pl.ds(16, 16)]
                b4[l, pl.ds(28, 16)] = b3[l, pl.ds(28, 16)]

            o1 = pltpu.async_copy(b1, o_hbm.at[b, :, pl.ds(0, 128)], s1)
            o2 = pltpu.async_copy(b2, o_hbm.at[b, :, pl.ds(128, 128)], s2)
            o3 = pltpu.async_copy(b4, o_hbm.at[b, :, pl.ds(256, 44)], s3)
            o1.wait()
            o2.wait()
            o3.wait()

    return gather_kernel(weight, w_tail, inputs)


# speedup vs baseline: 2.0093x; 1.2174x over previous
import functools

import jax
import jax.numpy as jnp
from jax import lax
from jax.experimental import pallas as pl
from jax.experimental.pallas import tpu as pltpu
from jax.experimental.pallas import tpu_sc as plsc

_B = 4096
_L = 50
_DIM = 300
_NW = 32
_BPW = _B // _NW
_BPS = 2
_NS = _BPW // _BPS
_R = _BPS * _L


def kernel(inputs, weight):
    w_tail = jnp.pad(weight[:, 256:_DIM], ((0, 0), (0, 84)))
    mesh = plsc.VectorSubcoreMesh(core_axis_name="c", subcore_axis_name="s")

    @functools.partial(
        pl.kernel,
        out_type=jax.ShapeDtypeStruct((_B, _L, _DIM), weight.dtype),
        mesh=mesh,
        scratch_types=[
            pltpu.VMEM((_BPS, _L), jnp.int32),
            pltpu.VMEM((_BPS, _L), jnp.int32),
            pltpu.VMEM((_R, 128), jnp.float32),
            pltpu.VMEM((_R, 128), jnp.float32),
            pltpu.VMEM((_R, 128), jnp.float32),
            pltpu.VMEM((_R, 128), jnp.float32),
            pltpu.VMEM((_R, 128), jnp.float32),
            pltpu.VMEM((_R, 128), jnp.float32),
            pltpu.VMEM((_R, 44), jnp.float32),
            pltpu.VMEM((_R, 44), jnp.float32),
            pltpu.SemaphoreType.DMA,
            pltpu.SemaphoreType.DMA,
            pltpu.SemaphoreType.DMA,
            pltpu.SemaphoreType.DMA,
            pltpu.SemaphoreType.DMA,
            pltpu.SemaphoreType.DMA,
        ],
    )
    def gather_kernel(w_hbm, wt_hbm, i_hbm, o_hbm,
                      iv0, iv1, b1_0, b1_1, b2_0, b2_1, b3_0, b3_1,
                      b4_0, b4_1, si0, si1, sg0, sg1, ss0, ss1):
        wid = lax.axis_index("s") * 2 + lax.axis_index("c")
        base = wid * _BPW
        iv = [iv0, iv1]
        b1 = [b1_0, b1_1]
        b2 = [b2_0, b2_1]
        b3 = [b3_0, b3_1]
        b4 = [b4_0, b4_1]
        si = [si0, si1]
        sg = [sg0, sg1]
        ss = [ss0, ss1]

        def batch_of(s):
            return base + s * _BPS

        def fire_idx(s, p):
            pltpu.async_copy(i_hbm.at[pl.ds(batch_of(s), _BPS)], iv[p], si[p])

        def wait_idx(s, p):
            pltpu.make_async_copy(
                i_hbm.at[pl.ds(batch_of(s), _BPS)], iv[p], si[p]).wait()

        def gather_args(p):
            out = []
            for k in range(_BPS):
                ik = iv[p].at[k]
                rows = pl.ds(k * _L, _L)
                out.append((w_hbm.at[:, pl.ds(0, 128)].at[ik], b1[p].at[rows]))
                out.append((w_hbm.at[:, pl.ds(128, 128)].at[ik], b2[p].at[rows]))
                out.append((wt_hbm.at[ik], b3[p].at[rows]))
            return out

        def fire_gathers(s, p):
            for src, dst in gather_args(p):
                pltpu.async_copy(src, dst, sg[p])

        def wait_gathers(s, p):
            for src, dst in gather_args(p):
                pltpu.make_async_copy(src, dst, sg[p]).wait()

        def tail_moves(p):
            src, dst = b3[p], b4[p]

            @pl.loop(0, _R)
            def _(l):
                dst[l, pl.ds(0, 16)] = src[l, pl.ds(0, 16)]
                dst[l, pl.ds(16, 16)] = src[l, pl.ds(16, 16)]
                dst[l, pl.ds(28, 16)] = src[l, pl.ds(28, 16)]

        def store_args(s, p):
            bb = batch_of(s)
            out = []
            for k in range(_BPS):
                rows = pl.ds(k * _L, _L)
                out.append((b1[p].at[rows], o_hbm.at[bb + k, :, pl.ds(0, 128)]))
                out.append((b2[p].at[rows], o_hbm.at[bb + k, :, pl.ds(128, 128)]))
                out.append((b4[p].at[rows], o_hbm.at[bb + k, :, pl.ds(256, 44)]))
            return out

        def fire_stores(s, p):
            for src, dst in store_args(s, p):
                pltpu.async_copy(src, dst, ss[p])

        def wait_stores(s, p):
            for src, dst in store_args(s, p):
                pltpu.make_async_copy(src, dst, ss[p]).wait()

        def leg(s, p, first_pair=False, fire_next=True, finish_prev=True):
            wait_idx(s, p)
            if not first_pair:
                wait_stores(s, p)
            fire_gathers(s, p)
            if fire_next:
                fire_idx(s + 1, 1 - p)
            if finish_prev:
                wait_gathers(s, 1 - p)
                tail_moves(1 - p)
                fire_stores(s - 1, 1 - p)

        fire_idx(0, 0)
        leg(0, 0, first_pair=True, finish_prev=False)
        leg(1, 1, first_pair=True)

        @pl.loop(0, (_NS - 4) // 2)
        def _(j):
            s = 2 + 2 * j
            leg(s, 0)
            leg(s + 1, 1)

        leg(_NS - 2, 0)
        leg(_NS - 1, 1, fire_next=False)

        wait_gathers(_NS - 1, 1)
        tail_moves(1)
        fire_stores(_NS - 1, 1)
        wait_stores(_NS - 2, 0)
        wait_stores(_NS - 1, 1)

    return gather_kernel(weight, w_tail, inputs)


# speedup vs baseline: 2.2289x; 1.1093x over previous
import dataclasses
import functools

import jax
import jax.numpy as jnp
from jax import lax
from jax.experimental import pallas as pl
from jax.experimental.pallas import tpu as pltpu
from jax.experimental.pallas import tpu_sc as plsc

_B = 4096
_L = 50
_DIM = 300
_V = 100000
_NW = 32
_DMAX = 10
_NC = _B // 16


def kernel(inputs, weight):
    w_t = weight.T
    i_t = inputs.T

    mesh = plsc.VectorSubcoreMesh(core_axis_name="c", subcore_axis_name="s")
    cp = pltpu.CompilerParams()
    if "needs_layout_passes" in pltpu.CompilerParams.__dataclass_fields__:
        cp = dataclasses.replace(cp, needs_layout_passes=False)

    @functools.partial(
        pl.kernel,
        out_type=jax.ShapeDtypeStruct((_L, _DIM, _B), weight.dtype),
        mesh=mesh,
        compiler_params=cp,
        scratch_types=[
            pltpu.VMEM((_V,), jnp.float32),
            pltpu.VMEM((_B,), jnp.int32),
            pltpu.VMEM((_B,), jnp.int32),
            pltpu.VMEM((_B,), jnp.float32),
            pltpu.VMEM((_B,), jnp.float32),
            pltpu.SemaphoreType.DMA,
            pltpu.SemaphoreType.DMA,
            pltpu.SemaphoreType.DMA,
            pltpu.SemaphoreType.DMA,
        ],
    )
    def gather_kernel(w_hbm, i_hbm, o_hbm, row_v,
                      iv0, iv1, ov0, ov1, si0, si1, ss0, ss1):
        sid = lax.axis_index("s")
        cid = lax.axis_index("c")
        wid = sid * 2 + cid
        iv = [iv0, iv1]
        ov = [ov0, ov1]
        si = [si0, si1]
        ss = [ss0, ss1]

        def fire_idx(l, p):
            pltpu.async_copy(i_hbm.at[l], iv[p], si[p])

        def wait_idx(l, p):
            pltpu.make_async_copy(i_hbm.at[l], iv[p], si[p]).wait()

        def fire_store(l, d, p):
            pltpu.async_copy(ov[p], o_hbm.at[l, d], ss[p])

        def wait_store(l, d, p):
            pltpu.make_async_copy(ov[p], o_hbm.at[l, d], ss[p]).wait()

        def compute(p):
            src, dst = iv[p], ov[p]

            @pl.loop(0, _NC, step=8)
            def _(c0):
                for k in range(8):
                    c = c0 + k
                    vidx = src[pl.ds(c * 16, 16)]
                    dst[pl.ds(c * 16, 16)] = plsc.load_gather(row_v, [vidx])

        @pl.loop(0, _DMAX)
        def _(i):
            d = wid + _NW * i

            @pl.when(d < _DIM)
            def _():
                pltpu.sync_copy(w_hbm.at[d], row_v)
                fire_idx(0, 0)

                def leg(l, p, first, fire_next):
                    wait_idx(l, p)
                    if fire_next:
                        fire_idx(l + 1, 1 - p)
                    if not first:
                        wait_store(l - 2, d, p)
                    compute(p)
                    fire_store(l, d, p)

                leg(0, 0, True, True)
                leg(1, 1, True, True)

                @pl.loop(0, (_L - 4) // 2)
                def _(j):
                    l = 2 + 2 * j
                    leg(l, 0, False, True)
                    leg(l + 1, 1, False, True)

                leg(_L - 2, 0, False, True)
                leg(_L - 1, 1, False, False)
                wait_store(_L - 2, d, 0)
                wait_store(_L - 1, d, 1)

    out = gather_kernel(w_t, i_t)
    return out.transpose(2, 0, 1)


# speedup vs baseline: 2.6286x; 1.1793x over previous
import dataclasses
import functools

import jax
import jax.numpy as jnp
from jax import lax
from jax.experimental import pallas as pl
from jax.experimental.pallas import tpu as pltpu
from jax.experimental.pallas import tpu_sc as plsc

_B = 4096
_L = 50
_DIM = 300
_V = 100000
_NW = 32
_DMAX = 10
_NC = _B // 16


def kernel(inputs, weight):
    w_t = weight.T
    i_t = inputs.T

    mesh = plsc.VectorSubcoreMesh(core_axis_name="c", subcore_axis_name="s")
    cp = pltpu.CompilerParams()
    if "needs_layout_passes" in pltpu.CompilerParams.__dataclass_fields__:
        cp = dataclasses.replace(cp, needs_layout_passes=False)

    @functools.partial(
        pl.kernel,
        out_type=jax.ShapeDtypeStruct((_L, _DIM, _B), weight.dtype),
        mesh=mesh,
        compiler_params=cp,
        scratch_types=[
            pltpu.VMEM((_V,), jnp.float32),
            pltpu.VMEM((_B,), jnp.int32),
            pltpu.VMEM((_B,), jnp.int32),
            pltpu.VMEM((_B,), jnp.float32),
            pltpu.VMEM((_B,), jnp.float32),
            pltpu.SemaphoreType.DMA,
            pltpu.SemaphoreType.DMA,
            pltpu.SemaphoreType.DMA,
            pltpu.SemaphoreType.DMA,
        ],
    )
    def gather_kernel(w_hbm, i_hbm, o_hbm, row_v,
                      iv0, iv1, ov0, ov1, si0, si1, ss0, ss1):
        sid = lax.axis_index("s")
        cid = lax.axis_index("c")
        wid = sid * 2 + cid
        iv = [iv0, iv1]
        ov = [ov0, ov1]
        si = [si0, si1]
        ss = [ss0, ss1]

        def fire_idx(l, p):
            pltpu.async_copy(i_hbm.at[l], iv[p], si[p])

        def wait_idx(l, p):
            pltpu.make_async_copy(i_hbm.at[l], iv[p], si[p]).wait()

        def fire_store(l, d, p):
            pltpu.async_copy(ov[p], o_hbm.at[l, d], ss[p])

        def wait_store(l, d, p):
            pltpu.make_async_copy(ov[p], o_hbm.at[l, d], ss[p]).wait()

        def compute(p):
            src, dst = iv[p], ov[p]

            @pl.loop(0, _NC, step=16)
            def _(c0):
                vals = []
                for k in range(16):
                    vidx = src[pl.ds((c0 + k) * 16, 16)]
                    vals.append(plsc.load_gather(row_v, [vidx]))
                for k in range(16):
                    dst[pl.ds((c0 + k) * 16, 16)] = vals[k]

        @pl.loop(0, _DMAX)
        def _(i):
            d = wid + _NW * i

            @pl.when(d < _DIM)
            def _():
                pltpu.sync_copy(w_hbm.at[d], row_v)
                fire_idx(0, 0)

                def leg(l, p, first, fire_next):
                    wait_idx(l, p)
                    if fire_next:
                        fire_idx(l + 1, 1 - p)
                    if not first:
                        wait_store(l - 2, d, p)
                    compute(p)
                    fire_store(l, d, p)

                leg(0, 0, True, True)
                leg(1, 1, True, True)

                @pl.loop(0, (_L - 4) // 2)
                def _(j):
                    l = 2 + 2 * j
                    leg(l, 0, False, True)
                    leg(l + 1, 1, False, True)

                leg(_L - 2, 0, False, True)
                leg(_L - 1, 1, False, False)
                wait_store(_L - 2, d, 0)
                wait_store(_L - 1, d, 1)

    out = gather_kernel(w_t, i_t)
    return out.transpose(2, 0, 1)


# speedup vs baseline: 2.6545x; 1.0098x over previous
import dataclasses
import functools

import jax
import jax.numpy as jnp
from jax import lax
from jax.experimental import pallas as pl
from jax.experimental.pallas import tpu as pltpu
from jax.experimental.pallas import tpu_sc as plsc

_B = 4096
_L = 50
_DIM = 300
_V = 100000
_NW = 32
_DMAX = 10
_NC = _B // 16


def kernel(inputs, weight):
    w_t = weight.T
    i_t = inputs.T

    mesh = plsc.VectorSubcoreMesh(core_axis_name="c", subcore_axis_name="s")
    cp = pltpu.CompilerParams()
    if "needs_layout_passes" in pltpu.CompilerParams.__dataclass_fields__:
        cp = dataclasses.replace(cp, needs_layout_passes=False)

    @functools.partial(
        pl.kernel,
        out_type=jax.ShapeDtypeStruct((_L, _DIM, _B), weight.dtype),
        mesh=mesh,
        compiler_params=cp,
        scratch_types=[
            pltpu.VMEM((_V,), jnp.float32),
            pltpu.VMEM((_B,), jnp.int32),
            pltpu.VMEM((_B,), jnp.int32),
            pltpu.VMEM((_B,), jnp.float32),
            pltpu.VMEM((_B,), jnp.float32),
            pltpu.SemaphoreType.DMA,
            pltpu.SemaphoreType.DMA,
            pltpu.SemaphoreType.DMA,
            pltpu.SemaphoreType.DMA,
        ],
    )
    def gather_kernel(w_hbm, i_hbm, o_hbm, row_v,
                      iv0, iv1, ov0, ov1, si0, si1, ss0, ss1):
        sid = lax.axis_index("s")
        cid = lax.axis_index("c")
        wid = sid * 2 + cid
        iv = [iv0, iv1]
        ov = [ov0, ov1]
        si = [si0, si1]
        ss = [ss0, ss1]

        def fire_idx(l, p):
            pltpu.async_copy(i_hbm.at[l], iv[p], si[p])

        def wait_idx(l, p):
            pltpu.make_async_copy(i_hbm.at[l], iv[p], si[p]).wait()

        def fire_store(l, d, p):
            pltpu.async_copy(ov[p], o_hbm.at[l, d], ss[p])

        def wait_store(l, d, p):
            pltpu.make_async_copy(ov[p], o_hbm.at[l, d], ss[p]).wait()

        def compute(p):
            src, dst = iv[p], ov[p]

            @pl.loop(0, _NC, step=32)
            def _(c0):
                vals = []
                for k in range(32):
                    vidx = src[pl.ds((c0 + k) * 16, 16)]
                    vals.append(plsc.load_gather(row_v, [vidx]))
                for k in range(32):
                    dst[pl.ds((c0 + k) * 16, 16)] = vals[k]

        @pl.loop(0, _DMAX)
        def _(i):
            d = wid + _NW * i

            @pl.when(d < _DIM)
            def _():
                fire_idx(0, 0)
                pltpu.sync_copy(w_hbm.at[d], row_v)

                def leg(l, p, first, fire_next):
                    wait_idx(l, p)
                    if fire_next:
                        fire_idx(l + 1, 1 - p)
                    if not first:
                        wait_store(l - 2, d, p)
                    compute(p)
                    fire_store(l, d, p)

                leg(0, 0, True, True)
                leg(1, 1, True, True)

                @pl.loop(0, (_L - 4) // 2)
                def _(j):
                    l = 2 + 2 * j
                    leg(l, 0, False, True)
                    leg(l + 1, 1, False, True)

                leg(_L - 2, 0, False, True)
                leg(_L - 1, 1, False, False)
                wait_store(_L - 2, d, 0)
                wait_store(_L - 1, d, 1)

    out = gather_kernel(w_t, i_t)
    return out.transpose(2, 0, 1)


# speedup vs baseline: 2.7345x; 1.0301x over previous
import dataclasses
import functools

import jax
import jax.numpy as jnp
from jax import lax
from jax.experimental import pallas as pl
from jax.experimental.pallas import tpu as pltpu
from jax.experimental.pallas import tpu_sc as plsc

_B = 4096
_L = 50
_DIM = 300
_V = 100000
_NW = 32
_DMAX = 10
_NC = _B // 16


def kernel(inputs, weight):
    w_t = weight.T
    i_t = inputs.T

    mesh = plsc.VectorSubcoreMesh(core_axis_name="c", subcore_axis_name="s")
    cp = pltpu.CompilerParams()
    if "needs_layout_passes" in pltpu.CompilerParams.__dataclass_fields__:
        cp = dataclasses.replace(cp, needs_layout_passes=False)

    @functools.partial(
        pl.kernel,
        out_type=jax.ShapeDtypeStruct((_L, _DIM, _B), weight.dtype),
        mesh=mesh,
        compiler_params=cp,
        scratch_types=[
            pltpu.VMEM((_V,), jnp.float32),
            pltpu.VMEM((_B,), jnp.int32),
            pltpu.VMEM((_B,), jnp.int32),
            pltpu.VMEM((_B,), jnp.float32),
            pltpu.VMEM((_B,), jnp.float32),
            pltpu.SemaphoreType.DMA,
            pltpu.SemaphoreType.DMA,
            pltpu.SemaphoreType.DMA,
            pltpu.SemaphoreType.DMA,
        ],
    )
    def gather_kernel(w_hbm, i_hbm, o_hbm, row_v,
                      iv0, iv1, ov0, ov1, si0, si1, ss0, ss1):
        sid = lax.axis_index("s")
        cid = lax.axis_index("c")
        wid = sid * 2 + cid
        iv = [iv0, iv1]
        ov = [ov0, ov1]
        si = [si0, si1]
        ss = [ss0, ss1]

        def fire_idx(l, p):
            pltpu.async_copy(i_hbm.at[l], iv[p], si[p])

        def wait_idx(l, p):
            pltpu.make_async_copy(i_hbm.at[l], iv[p], si[p]).wait()

        def fire_store(l, d, p):
            pltpu.async_copy(ov[p], o_hbm.at[l, d], ss[p])

        def wait_store(l, d, p):
            pltpu.make_async_copy(ov[p], o_hbm.at[l, d], ss[p]).wait()

        def compute(p):
            src, dst = iv[p], ov[p]

            @pl.loop(0, _NC, step=32)
            def _(c0):
                vals = []
                for k in range(32):
                    vidx = src[pl.ds((c0 + k) * 16, 16)]
                    vals.append(plsc.load_gather(row_v, [vidx]))
                for k in range(32):
                    dst[pl.ds((c0 + k) * 16, 16)] = vals[k]

        def run_rows(d, l0, nl):
            fire_idx(l0, 0)
            pltpu.sync_copy(w_hbm.at[d], row_v)

            def leg(l, p, first, fire_next):
                wait_idx(l, p)
                if fire_next:
                    fire_idx(l + 1, 1 - p)
                if not first:
                    wait_store(l - 2, d, p)
                compute(p)
                fire_store(l, d, p)

            leg(l0, 0, True, True)
            leg(l0 + 1, 1, True, True)

            npeel = 2 if nl % 2 == 0 else 3

            @pl.loop(0, (nl - 2 - npeel) // 2)
            def _(j):
                l = l0 + 2 + 2 * j
                leg(l, 0, False, True)
                leg(l + 1, 1, False, True)

            if npeel == 3:
                leg(l0 + nl - 3, 0, False, True)
            leg(l0 + nl - 2, nl % 2, False, True)
            leg(l0 + nl - 1, 1 - (nl % 2), False, False)
            wait_store(l0 + nl - 2, d, nl % 2)
            wait_store(l0 + nl - 1, d, 1 - (nl % 2))

        @pl.loop(0, _DIM // _NW)
        def _(i):
            run_rows(wid + _NW * i, 0, _L)

        @pl.when(wid < 2 * (_DIM % _NW))
        def _():
            d = (_DIM // _NW) * _NW + wid // 2
            l0 = (wid % 2) * (_L // 2)
            run_rows(d, l0, _L // 2)

    out = gather_kernel(w_t, i_t)
    return out.transpose(2, 0, 1)
